# Initial kernel scaffold; baseline (speedup 1.0000x reference)
import jax
import jax.numpy as jnp
from jax.experimental import pallas as pl


def kernel(x, edge_index, W1l, W1r, b1, W2l, W2r, b2, Wp, bp, Wc, bc):
    raise NotImplementedError("write your pallas kernel here")


# speedup vs baseline: 7.5646x; 7.5646x over previous
import functools

import jax
import jax.numpy as jnp
from jax import lax
from jax.experimental import pallas as pl
from jax.experimental.pallas import tpu as pltpu
from jax.experimental.pallas import tpu_sc as plsc

NC = 2
NS = 16
NW = NC * NS
CB = 128


def _round_up(a, b):
    return (a + b - 1) // b * b


def _make_segsum(Npad, D, J):
    mesh = plsc.VectorSubcoreMesh(core_axis_name="c", subcore_axis_name="s")
    rows_per_tile = Npad // NS
    ZR = 128
    n_zero_copies = rows_per_tile // ZR

    @functools.partial(
        pl.kernel,
        mesh=mesh,
        out_type=jax.ShapeDtypeStruct((NC, Npad, D), jnp.float32),
        scratch_types=[
            pltpu.VMEM((J, CB), jnp.int32),
            pltpu.VMEM((J, CB), jnp.int32),
            pltpu.VMEM((CB, D), jnp.float32),
            pltpu.VMEM((ZR, D), jnp.float32),
            pltpu.VMEM_SHARED((Npad, D), jnp.float32),
            pltpu.SemaphoreType.DMA,
        ],
        compiler_params=pltpu.CompilerParams(use_tc_tiling_on_sc=False),
    )
    def segsum(feat_hbm, src_hbm, dst_hbm, out_hbm, src_v, dst_v, rows_v,
               zb_v, acc_sh, sem):
        c = lax.axis_index("c")
        s = lax.axis_index("s")
        wid = s * NC + c

        pltpu.sync_copy(src_hbm.at[wid], src_v)
        pltpu.sync_copy(dst_hbm.at[wid], dst_v)

        zeros16 = jnp.zeros((16,), jnp.float32)

        def zero_row(i, _):
            for cc in range(D // 16):
                zb_v[i, pl.ds(cc * 16, 16)] = zeros16
            return 0

        lax.fori_loop(0, ZR, zero_row, 0)
        for k in range(n_zero_copies):
            pltpu.sync_copy(
                zb_v, acc_sh.at[pl.ds(s * rows_per_tile + k * ZR, ZR)])
        plsc.subcore_barrier()

        def edge_batch(j, _):
            pltpu.async_copy(feat_hbm.at[src_v.at[j]], rows_v, sem).wait()
            pltpu.sync_copy(rows_v, acc_sh.at[dst_v.at[j]], add=True)
            return 0

        lax.fori_loop(0, J, edge_batch, 0)
        plsc.subcore_barrier()

        pltpu.sync_copy(
            acc_sh.at[pl.ds(s * rows_per_tile, rows_per_tile)],
            out_hbm.at[c, pl.ds(s * rows_per_tile, rows_per_tile)])

    return segsum


def _tc_pre(x, W1l, W1r, b1, Npad, D1, R):
    N, DIN = x.shape
    H = W1l.shape[1]

    def body(x_ref, wl_ref, wr_ref, b_ref, xl_ref, xr_ref):
        xb = x_ref[...]
        y = jnp.dot(xb, wl_ref[...], preferred_element_type=jnp.float32)
        ones = jnp.ones((R, 1), jnp.float32)
        zpad = jnp.zeros((R, D1 - H - 1), jnp.float32)
        xl_ref[...] = jnp.concatenate([y, ones, zpad], axis=1)
        xr_ref[...] = (
            jnp.dot(xb, wr_ref[...], preferred_element_type=jnp.float32)
            + b_ref[...])

    grid = (Npad // R,)
    return pl.pallas_call(
        body,
        grid=grid,
        in_specs=[
            pl.BlockSpec((R, DIN), lambda i: (i, 0)),
            pl.BlockSpec((DIN, H), lambda i: (0, 0)),
            pl.BlockSpec((DIN, H), lambda i: (0, 0)),
            pl.BlockSpec((1, H), lambda i: (0, 0)),
        ],
        out_specs=[
            pl.BlockSpec((R, D1), lambda i: (i, 0)),
            pl.BlockSpec((R, H), lambda i: (i, 0)),
        ],
        out_shape=[
            jax.ShapeDtypeStruct((Npad, D1), jnp.float32),
            jax.ShapeDtypeStruct((Npad, H), jnp.float32),
        ],
    )(x, W1l, W1r, b1.reshape(1, H))


def _tc_mid(part1, xr1, W2l, W2r, b2, Npad, D1, R):
    H = xr1.shape[1]
    H2 = W2l.shape[1]

    def body(p_ref, xr_ref, wl_ref, wr_ref, b_ref, hl_ref, hr_ref, dg_ref):
        p = p_ref[...]
        agg = p[0] + p[1]
        deg = jnp.maximum(agg[:, H:H + 1], 1.0)
        h1 = jnp.maximum(agg[:, :H] / deg + xr_ref[...], 0.0)
        hl_ref[...] = jnp.dot(h1, wl_ref[...],
                              preferred_element_type=jnp.float32)
        hr_ref[...] = (
            jnp.dot(h1, wr_ref[...], preferred_element_type=jnp.float32)
            + b_ref[...])
        dg_ref[...] = jnp.broadcast_to(deg, (R, 8))

    grid = (Npad // R,)
    return pl.pallas_call(
        body,
        grid=grid,
        in_specs=[
            pl.BlockSpec((NC, R, D1), lambda i: (0, i, 0)),
            pl.BlockSpec((R, H), lambda i: (i, 0)),
            pl.BlockSpec((H, H2), lambda i: (0, 0)),
            pl.BlockSpec((H, H2), lambda i: (0, 0)),
            pl.BlockSpec((1, H2), lambda i: (0, 0)),
        ],
        out_specs=[
            pl.BlockSpec((R, H2), lambda i: (i, 0)),
            pl.BlockSpec((R, H2), lambda i: (i, 0)),
            pl.BlockSpec((R, 8), lambda i: (i, 0)),
        ],
        out_shape=[
            jax.ShapeDtypeStruct((Npad, H2), jnp.float32),
            jax.ShapeDtypeStruct((Npad, H2), jnp.float32),
            jax.ShapeDtypeStruct((Npad, 8), jnp.float32),
        ],
    )(part1, xr1, W2l, W2r, b2.reshape(1, H2))


def _tc_post(part2, hr2, degb, Wp, bp, Wc, bc, Npad, R):
    H2 = hr2.shape[1]
    OC = Wc.shape[1]
    OCp = 8

    Wc_p = jnp.zeros((H2, OCp), jnp.float32).at[:, :OC].set(Wc)
    bc_p = jnp.zeros((1, OCp), jnp.float32).at[0, :OC].set(bc)

    def body(p_ref, hr_ref, dg_ref, wp_ref, bp_ref, wc_ref, bc_ref,
             z_ref, l_ref):
        p = p_ref[...]
        agg = p[0] + p[1]
        deg = dg_ref[...][:, 0:1]
        h2 = jnp.maximum(agg / deg + hr_ref[...], 0.0)
        z = jnp.dot(h2, wp_ref[...], preferred_element_type=jnp.float32) \
            + bp_ref[...]
        z_ref[...] = z
        l_ref[...] = jnp.dot(z, wc_ref[...],
                             preferred_element_type=jnp.float32) + bc_ref[...]

    grid = (Npad // R,)
    return pl.pallas_call(
        body,
        grid=grid,
        in_specs=[
            pl.BlockSpec((NC, R, H2), lambda i: (0, i, 0)),
            pl.BlockSpec((R, H2), lambda i: (i, 0)),
            pl.BlockSpec((R, 8), lambda i: (i, 0)),
            pl.BlockSpec((H2, H2), lambda i: (0, 0)),
            pl.BlockSpec((1, H2), lambda i: (0, 0)),
            pl.BlockSpec((H2, OCp), lambda i: (0, 0)),
            pl.BlockSpec((1, OCp), lambda i: (0, 0)),
        ],
        out_specs=[
            pl.BlockSpec((R, H2), lambda i: (i, 0)),
            pl.BlockSpec((R, OCp), lambda i: (i, 0)),
        ],
        out_shape=[
            jax.ShapeDtypeStruct((Npad, H2), jnp.float32),
            jax.ShapeDtypeStruct((Npad, OCp), jnp.float32),
        ],
    )(part2, hr2, degb, Wp, bp.reshape(1, H2), Wc_p, bc_p)


def kernel(x, edge_index, W1l, W1r, b1, W2l, W2r, b2, Wp, bp, Wc, bc):
    N, DIN = x.shape
    E = edge_index.shape[1]
    H = W1l.shape[1]
    H2 = W2l.shape[1]
    OC = Wc.shape[1]

    R = 1024
    Npad = _round_up(N + 1, R)
    D1 = _round_up(H + 1, 16)
    J = _round_up(E, NW * CB) // (NW * CB)
    Epad = J * NW * CB

    xp = jnp.pad(x, ((0, Npad - N), (0, 0)))
    src = jnp.pad(edge_index[0], (0, Epad - E), constant_values=N)
    dst = jnp.pad(edge_index[1], (0, Epad - E), constant_values=N)
    src_r = src.reshape(NW, J, CB)
    dst_r = dst.reshape(NW, J, CB)

    xl1e, xr1 = _tc_pre(xp, W1l, W1r, b1, Npad, D1, R)

    part1 = _make_segsum(Npad, D1, J)(xl1e, src_r, dst_r)

    hl2, hr2, degb = _tc_mid(part1, xr1, W2l, W2r, b2, Npad, D1, R)

    part2 = _make_segsum(Npad, H2, J)(hl2, src_r, dst_r)

    z, logits = _tc_post(part2, hr2, degb, Wp, bp, Wc, bc, Npad, R)

    return (logits[:N, :OC], z[:N, :H2])
